# baseline (speedup 1.0000x reference)
import functools

import jax
import jax.numpy as jnp
from jax import lax
from jax.experimental import pallas as pl
from jax.experimental.pallas import tpu as pltpu
from jax.experimental.pallas import tpu_sc as plsc

B, N, DIM = 16, 2048, 3
SUB = 128
EMBED = 3 * SUB
RB = 256
NW = 32
RPW = B * N // NW
L = 16

_INF = float("inf")


def _sc_top4_body(px_hbm, py_hbm, pz_hbm, out_hbm, px_v, py_v, pz_v, ob_v):
    wid = lax.axis_index("s") * 2 + lax.axis_index("c")
    b = wid // 2
    row0 = jnp.where(wid % 2 == 0, 0, RPW)

    pltpu.sync_copy(px_hbm.at[pl.ds(b * N, N)], px_v)
    pltpu.sync_copy(py_hbm.at[pl.ds(b * N, N)], py_v)
    pltpu.sync_copy(pz_hbm.at[pl.ds(b * N, N)], pz_v)

    inf16 = jnp.full((L,), _INF, jnp.float32)
    lane = lax.iota(jnp.int32, L)

    def group_body(g, _):
        base = row0 + g * L
        sl = pl.ds(base, L)
        xr = px_v[sl]
        yr = py_v[sl]
        zr = pz_v[sl]

        def col_body(j, carry):
            m1, m2, m3, m4, jv = carry
            a = plsc.load_gather(px_v, [jv])
            bb = plsc.load_gather(py_v, [jv])
            c = plsc.load_gather(pz_v, [jv])
            dx = a - xr
            dy = bb - yr
            dz = c - zr
            t = dx * dx + dy * dy + dz * dz
            h = jnp.maximum(t, m1)
            m1 = jnp.minimum(t, m1)
            h2 = jnp.maximum(h, m2)
            m2 = jnp.minimum(h, m2)
            h3 = jnp.maximum(h2, m3)
            m3 = jnp.minimum(h2, m3)
            m4 = jnp.minimum(h3, m4)
            return m1, m2, m3, m4, jv + 1

        zero = jnp.zeros((L,), jnp.int32)
        _, m2, m3, m4, _ = lax.fori_loop(
            0, N, col_body, (inf16, inf16, inf16, inf16, zero))

        lidx = ((g * L) + lane) * 4
        plsc.store_scatter(ob_v, [lidx], m2)
        plsc.store_scatter(ob_v, [lidx + 1], m3)
        plsc.store_scatter(ob_v, [lidx + 2], m4)
        return 0

    lax.fori_loop(0, RPW // L, group_body, 0)

    pltpu.sync_copy(ob_v, out_hbm.at[pl.ds((b * N + row0) * 4, RPW * 4)])


@functools.partial(
    pl.kernel,
    mesh=plsc.VectorSubcoreMesh(core_axis_name="c", subcore_axis_name="s"),
    out_type=jax.ShapeDtypeStruct((B * N * 4,), jnp.float32),
    scratch_types=[
        pltpu.VMEM((N,), jnp.float32),
        pltpu.VMEM((N,), jnp.float32),
        pltpu.VMEM((N,), jnp.float32),
        pltpu.VMEM((RPW * 4,), jnp.float32),
    ],
    compiler_params=pltpu.CompilerParams(needs_layout_passes=False),
)
def _sc_top4(px_hbm, py_hbm, pz_hbm, out_hbm, px_v, py_v, pz_v, ob_v):
    _sc_top4_body(px_hbm, py_hbm, pz_hbm, out_hbm, px_v, py_v, pz_v, ob_v)


def _tc_body(pts_ref, ptsT_ref, top_ref, wrel_ref, brel_ref, wdist_ref,
             bdist_ref, wdens_ref, bdens_ref, wout_ref, bout_ref, out_ref):
    pts_blk = pts_ref[0]
    ptsT = ptsT_ref[0]

    csum = jnp.sum(ptsT, axis=1)
    centroid = (csum / jnp.float32(N))[None, :]
    rel = pts_blk - centroid

    rel_f = lax.dot_general(rel, wrel_ref[...],
                            (((1,), (0,)), ((), ())),
                            preferred_element_type=jnp.float32)
    rel_f = rel_f + brel_ref[...]

    cdist = jnp.sqrt(jnp.sum(rel * rel, axis=1, keepdims=True))
    dist_f = cdist * wdist_ref[...] + bdist_ref[...]

    t = top_ref[0]
    v1 = jnp.maximum(t[:, 0:1], 0.0)
    v2 = jnp.maximum(t[:, 1:2], 0.0)
    v3 = jnp.maximum(t[:, 2:3], 0.0)
    density = (jnp.sqrt(v1) + jnp.sqrt(v2) + jnp.sqrt(v3)) / 3.0

    dens_f = density * wdens_ref[...] + bdens_ref[...]

    feat = jnp.concatenate([rel_f, dist_f, dens_f], axis=1)
    out = lax.dot_general(feat, wout_ref[...],
                          (((1,), (0,)), ((), ())),
                          preferred_element_type=jnp.float32)
    out_ref[0] = out + bout_ref[...]


@jax.jit
def kernel(points, W_rel, b_rel, W_dist, b_dist, W_dens, b_dens, W_out, b_out):
    pointsT = jnp.transpose(points, (0, 2, 1))
    px = pointsT[:, 0, :].reshape(B * N)
    py = pointsT[:, 1, :].reshape(B * N)
    pz = pointsT[:, 2, :].reshape(B * N)
    top4 = _sc_top4(px, py, pz).reshape(B, N, 4)
    grid = (B, N // RB)
    out = pl.pallas_call(
        _tc_body,
        grid=grid,
        in_specs=[
            pl.BlockSpec((1, RB, DIM), lambda b, r: (b, r, 0)),
            pl.BlockSpec((1, DIM, N), lambda b, r: (b, 0, 0)),
            pl.BlockSpec((1, RB, 4), lambda b, r: (b, r, 0)),
            pl.BlockSpec((DIM, SUB), lambda b, r: (0, 0)),
            pl.BlockSpec((1, SUB), lambda b, r: (0, 0)),
            pl.BlockSpec((1, SUB), lambda b, r: (0, 0)),
            pl.BlockSpec((1, SUB), lambda b, r: (0, 0)),
            pl.BlockSpec((1, SUB), lambda b, r: (0, 0)),
            pl.BlockSpec((1, SUB), lambda b, r: (0, 0)),
            pl.BlockSpec((EMBED, EMBED), lambda b, r: (0, 0)),
            pl.BlockSpec((1, EMBED), lambda b, r: (0, 0)),
        ],
        out_specs=pl.BlockSpec((1, RB, EMBED), lambda b, r: (b, r, 0)),
        out_shape=jax.ShapeDtypeStruct((B, N, EMBED), jnp.float32),
        compiler_params=pltpu.CompilerParams(
            dimension_semantics=("parallel", "arbitrary"),
        ),
    )(points, pointsT, top4,
      W_rel, b_rel[None, :], W_dist, b_dist[None, :],
      W_dens, b_dens[None, :], W_out, b_out[None, :])
    return out


# speedup vs baseline: 1.5065x; 1.5065x over previous
import functools

import jax
import jax.numpy as jnp
from jax import lax
from jax.experimental import pallas as pl
from jax.experimental.pallas import tpu as pltpu
from jax.experimental.pallas import tpu_sc as plsc

B, N, DIM = 16, 2048, 3
SUB = 128
EMBED = 3 * SUB
RB = 256
NW = 32
RPW = B * N // NW
L = 16

_INF = float("inf")


def _sc_top4_body(px_hbm, py_hbm, pz_hbm, out_hbm, px_v, py_v, pz_v, ob_v):
    wid = lax.axis_index("s") * 2 + lax.axis_index("c")
    b = wid // 2
    row0 = jnp.where(wid % 2 == 0, 0, RPW)

    pltpu.sync_copy(px_hbm.at[pl.ds(b * N, N)], px_v)
    pltpu.sync_copy(py_hbm.at[pl.ds(b * N, N)], py_v)
    pltpu.sync_copy(pz_hbm.at[pl.ds(b * N, N)], pz_v)

    inf16 = jnp.full((L,), _INF, jnp.float32)
    lane = lax.iota(jnp.int32, L)
    G = 4

    def group_body(g, _):
        base = row0 + g * (G * L)
        xr = [px_v[pl.ds(base + k * L, L)] for k in range(G)]
        yr = [py_v[pl.ds(base + k * L, L)] for k in range(G)]
        zr = [pz_v[pl.ds(base + k * L, L)] for k in range(G)]

        def col_body(j, carry):
            ms, jv = carry
            a = plsc.load_gather(px_v, [jv])
            bb = plsc.load_gather(py_v, [jv])
            c = plsc.load_gather(pz_v, [jv])
            out = []
            for k in range(G):
                m1, m2, m3, m4 = ms[k]
                dx = a - xr[k]
                dy = bb - yr[k]
                dz = c - zr[k]
                t = dx * dx + dy * dy + dz * dz
                h = jnp.maximum(t, m1)
                m1 = jnp.minimum(t, m1)
                h2 = jnp.maximum(h, m2)
                m2 = jnp.minimum(h, m2)
                h3 = jnp.maximum(h2, m3)
                m3 = jnp.minimum(h2, m3)
                m4 = jnp.minimum(h3, m4)
                out.append((m1, m2, m3, m4))
            return tuple(out), jv + 1

        zero = jnp.zeros((L,), jnp.int32)
        init = tuple((inf16, inf16, inf16, inf16) for _ in range(G))
        ms, _ = lax.fori_loop(0, N, col_body, (init, zero))

        for k in range(G):
            _, m2, m3, m4 = ms[k]
            lidx = (g * (G * L) + k * L + lane) * 4
            plsc.store_scatter(ob_v, [lidx], m2)
            plsc.store_scatter(ob_v, [lidx + 1], m3)
            plsc.store_scatter(ob_v, [lidx + 2], m4)
        return 0

    lax.fori_loop(0, RPW // (G * L), group_body, 0)

    pltpu.sync_copy(ob_v, out_hbm.at[pl.ds((b * N + row0) * 4, RPW * 4)])


@functools.partial(
    pl.kernel,
    mesh=plsc.VectorSubcoreMesh(core_axis_name="c", subcore_axis_name="s"),
    out_type=jax.ShapeDtypeStruct((B * N * 4,), jnp.float32),
    scratch_types=[
        pltpu.VMEM((N,), jnp.float32),
        pltpu.VMEM((N,), jnp.float32),
        pltpu.VMEM((N,), jnp.float32),
        pltpu.VMEM((RPW * 4,), jnp.float32),
    ],
    compiler_params=pltpu.CompilerParams(needs_layout_passes=False),
)
def _sc_top4(px_hbm, py_hbm, pz_hbm, out_hbm, px_v, py_v, pz_v, ob_v):
    _sc_top4_body(px_hbm, py_hbm, pz_hbm, out_hbm, px_v, py_v, pz_v, ob_v)


def _tc_body(pts_ref, ptsT_ref, top_ref, wrel_ref, brel_ref, wdist_ref,
             bdist_ref, wdens_ref, bdens_ref, wout_ref, bout_ref, out_ref):
    pts_blk = pts_ref[0]
    ptsT = ptsT_ref[0]

    csum = jnp.sum(ptsT, axis=1)
    centroid = (csum / jnp.float32(N))[None, :]
    rel = pts_blk - centroid

    rel_f = lax.dot_general(rel, wrel_ref[...],
                            (((1,), (0,)), ((), ())),
                            preferred_element_type=jnp.float32)
    rel_f = rel_f + brel_ref[...]

    cdist = jnp.sqrt(jnp.sum(rel * rel, axis=1, keepdims=True))
    dist_f = cdist * wdist_ref[...] + bdist_ref[...]

    t = top_ref[0]
    v1 = jnp.maximum(t[:, 0:1], 0.0)
    v2 = jnp.maximum(t[:, 1:2], 0.0)
    v3 = jnp.maximum(t[:, 2:3], 0.0)
    density = (jnp.sqrt(v1) + jnp.sqrt(v2) + jnp.sqrt(v3)) / 3.0

    dens_f = density * wdens_ref[...] + bdens_ref[...]

    feat = jnp.concatenate([rel_f, dist_f, dens_f], axis=1)
    out = lax.dot_general(feat, wout_ref[...],
                          (((1,), (0,)), ((), ())),
                          preferred_element_type=jnp.float32)
    out_ref[0] = out + bout_ref[...]


@jax.jit
def kernel(points, W_rel, b_rel, W_dist, b_dist, W_dens, b_dens, W_out, b_out):
    pointsT = jnp.transpose(points, (0, 2, 1))
    px = pointsT[:, 0, :].reshape(B * N)
    py = pointsT[:, 1, :].reshape(B * N)
    pz = pointsT[:, 2, :].reshape(B * N)
    top4 = _sc_top4(px, py, pz).reshape(B, N, 4)
    grid = (B, N // RB)
    out = pl.pallas_call(
        _tc_body,
        grid=grid,
        in_specs=[
            pl.BlockSpec((1, RB, DIM), lambda b, r: (b, r, 0)),
            pl.BlockSpec((1, DIM, N), lambda b, r: (b, 0, 0)),
            pl.BlockSpec((1, RB, 4), lambda b, r: (b, r, 0)),
            pl.BlockSpec((DIM, SUB), lambda b, r: (0, 0)),
            pl.BlockSpec((1, SUB), lambda b, r: (0, 0)),
            pl.BlockSpec((1, SUB), lambda b, r: (0, 0)),
            pl.BlockSpec((1, SUB), lambda b, r: (0, 0)),
            pl.BlockSpec((1, SUB), lambda b, r: (0, 0)),
            pl.BlockSpec((1, SUB), lambda b, r: (0, 0)),
            pl.BlockSpec((EMBED, EMBED), lambda b, r: (0, 0)),
            pl.BlockSpec((1, EMBED), lambda b, r: (0, 0)),
        ],
        out_specs=pl.BlockSpec((1, RB, EMBED), lambda b, r: (b, r, 0)),
        out_shape=jax.ShapeDtypeStruct((B, N, EMBED), jnp.float32),
        compiler_params=pltpu.CompilerParams(
            dimension_semantics=("parallel", "arbitrary"),
        ),
    )(points, pointsT, top4,
      W_rel, b_rel[None, :], W_dist, b_dist[None, :],
      W_dens, b_dens[None, :], W_out, b_out[None, :])
    return out


# speedup vs baseline: 1.6100x; 1.0687x over previous
import functools

import jax
import jax.numpy as jnp
from jax import lax
from jax.experimental import pallas as pl
from jax.experimental.pallas import tpu as pltpu
from jax.experimental.pallas import tpu_sc as plsc

B, N, DIM = 16, 2048, 3
SUB = 128
EMBED = 3 * SUB
RB = 256
NW = 32
RPW = B * N // NW
L = 16
G = 4

_INF = float("inf")


def _sc_top3_body(px_hbm, py_hbm, pz_hbm, s2_hbm, out_hbm,
                  px_v, py_v, pz_v, s2_v, ob_v):
    wid = lax.axis_index("s") * 2 + lax.axis_index("c")
    b = wid // 2
    row0 = jnp.where(wid % 2 == 0, 0, RPW)

    pltpu.sync_copy(px_hbm.at[pl.ds(b * N, N)], px_v)
    pltpu.sync_copy(py_hbm.at[pl.ds(b * N, N)], py_v)
    pltpu.sync_copy(pz_hbm.at[pl.ds(b * N, N)], pz_v)
    pltpu.sync_copy(s2_hbm.at[pl.ds(b * N, N)], s2_v)

    inf16 = jnp.full((L,), _INF, jnp.float32)
    lane = lax.iota(jnp.int32, L)

    def group_body(g, _):
        base = row0 + g * (G * L)
        n2x = [px_v[pl.ds(base + k * L, L)] * -2.0 for k in range(G)]
        n2y = [py_v[pl.ds(base + k * L, L)] * -2.0 for k in range(G)]
        n2z = [pz_v[pl.ds(base + k * L, L)] * -2.0 for k in range(G)]
        riv = [base + k * L + lane for k in range(G)]

        def col_body(j, carry):
            ms, jv = carry
            a = plsc.load_gather(px_v, [jv])
            bb = plsc.load_gather(py_v, [jv])
            c = plsc.load_gather(pz_v, [jv])
            sj = plsc.load_gather(s2_v, [jv])
            out = []
            for k in range(G):
                m1, m2, m3 = ms[k]
                t = sj + a * n2x[k]
                t = t + bb * n2y[k]
                t = t + c * n2z[k]
                t = jnp.where(jv == riv[k], _INF, t)
                h = jnp.maximum(t, m1)
                m1 = jnp.minimum(t, m1)
                h2 = jnp.maximum(h, m2)
                m2 = jnp.minimum(h, m2)
                m3 = jnp.minimum(h2, m3)
                out.append((m1, m2, m3))
            return tuple(out), jv + 1

        zero = jnp.zeros((L,), jnp.int32)
        init = tuple((inf16, inf16, inf16) for _ in range(G))
        ms, _ = lax.fori_loop(0, N, col_body, (init, zero))

        for k in range(G):
            m1, m2, m3 = ms[k]
            sr = s2_v[pl.ds(base + k * L, L)]
            lidx = (g * (G * L) + k * L + lane) * 4
            plsc.store_scatter(ob_v, [lidx], m1 + sr)
            plsc.store_scatter(ob_v, [lidx + 1], m2 + sr)
            plsc.store_scatter(ob_v, [lidx + 2], m3 + sr)
        return 0

    lax.fori_loop(0, RPW // (G * L), group_body, 0)

    pltpu.sync_copy(ob_v, out_hbm.at[pl.ds((b * N + row0) * 4, RPW * 4)])


@functools.partial(
    pl.kernel,
    mesh=plsc.VectorSubcoreMesh(core_axis_name="c", subcore_axis_name="s"),
    out_type=jax.ShapeDtypeStruct((B * N * 4,), jnp.float32),
    scratch_types=[
        pltpu.VMEM((N,), jnp.float32),
        pltpu.VMEM((N,), jnp.float32),
        pltpu.VMEM((N,), jnp.float32),
        pltpu.VMEM((N,), jnp.float32),
        pltpu.VMEM((RPW * 4,), jnp.float32),
    ],
    compiler_params=pltpu.CompilerParams(needs_layout_passes=False),
)
def _sc_top3(px_hbm, py_hbm, pz_hbm, s2_hbm, out_hbm,
             px_v, py_v, pz_v, s2_v, ob_v):
    _sc_top3_body(px_hbm, py_hbm, pz_hbm, s2_hbm, out_hbm,
                  px_v, py_v, pz_v, s2_v, ob_v)


def _tc_body(pts_ref, ptsT_ref, top_ref, wrel_ref, brel_ref, wdist_ref,
             bdist_ref, wdens_ref, bdens_ref, wout_ref, bout_ref, out_ref):
    pts_blk = pts_ref[0]
    ptsT = ptsT_ref[0]

    csum = jnp.sum(ptsT, axis=1)
    centroid = (csum / jnp.float32(N))[None, :]
    rel = pts_blk - centroid

    rel_f = lax.dot_general(rel, wrel_ref[...],
                            (((1,), (0,)), ((), ())),
                            preferred_element_type=jnp.float32)
    rel_f = rel_f + brel_ref[...]

    cdist = jnp.sqrt(jnp.sum(rel * rel, axis=1, keepdims=True))
    dist_f = cdist * wdist_ref[...] + bdist_ref[...]

    t = top_ref[0]
    v1 = jnp.maximum(t[:, 0:1], 0.0)
    v2 = jnp.maximum(t[:, 1:2], 0.0)
    v3 = jnp.maximum(t[:, 2:3], 0.0)
    density = (jnp.sqrt(v1) + jnp.sqrt(v2) + jnp.sqrt(v3)) / 3.0

    dens_f = density * wdens_ref[...] + bdens_ref[...]

    feat = jnp.concatenate([rel_f, dist_f, dens_f], axis=1)
    out = lax.dot_general(feat, wout_ref[...],
                          (((1,), (0,)), ((), ())),
                          preferred_element_type=jnp.float32)
    out_ref[0] = out + bout_ref[...]


@jax.jit
def kernel(points, W_rel, b_rel, W_dist, b_dist, W_dens, b_dens, W_out, b_out):
    pointsT = jnp.transpose(points, (0, 2, 1))
    pointsTb = pointsT.astype(jnp.bfloat16).astype(jnp.float32)
    px = pointsTb[:, 0, :].reshape(B * N)
    py = pointsTb[:, 1, :].reshape(B * N)
    pz = pointsTb[:, 2, :].reshape(B * N)
    s2 = jnp.sum(pointsT * pointsT, axis=1).reshape(B * N)
    top3 = _sc_top3(px, py, pz, s2).reshape(B, N, 4)
    grid = (B, N // RB)
    out = pl.pallas_call(
        _tc_body,
        grid=grid,
        in_specs=[
            pl.BlockSpec((1, RB, DIM), lambda b, r: (b, r, 0)),
            pl.BlockSpec((1, DIM, N), lambda b, r: (b, 0, 0)),
            pl.BlockSpec((1, RB, 4), lambda b, r: (b, r, 0)),
            pl.BlockSpec((DIM, SUB), lambda b, r: (0, 0)),
            pl.BlockSpec((1, SUB), lambda b, r: (0, 0)),
            pl.BlockSpec((1, SUB), lambda b, r: (0, 0)),
            pl.BlockSpec((1, SUB), lambda b, r: (0, 0)),
            pl.BlockSpec((1, SUB), lambda b, r: (0, 0)),
            pl.BlockSpec((1, SUB), lambda b, r: (0, 0)),
            pl.BlockSpec((EMBED, EMBED), lambda b, r: (0, 0)),
            pl.BlockSpec((1, EMBED), lambda b, r: (0, 0)),
        ],
        out_specs=pl.BlockSpec((1, RB, EMBED), lambda b, r: (b, r, 0)),
        out_shape=jax.ShapeDtypeStruct((B, N, EMBED), jnp.float32),
        compiler_params=pltpu.CompilerParams(
            dimension_semantics=("parallel", "arbitrary"),
        ),
    )(points, pointsT, top3,
      W_rel, b_rel[None, :], W_dist, b_dist[None, :],
      W_dens, b_dens[None, :], W_out, b_out[None, :])
    return out


# speedup vs baseline: 1.6469x; 1.0229x over previous
import functools

import jax
import jax.numpy as jnp
from jax import lax
from jax.experimental import pallas as pl
from jax.experimental.pallas import tpu as pltpu
from jax.experimental.pallas import tpu_sc as plsc

B, N, DIM = 16, 2048, 3
SUB = 128
EMBED = 3 * SUB
RB = 256
NW = 32
RPW = B * N // NW
L = 16
G = 4

_INF = float("inf")


def _round_to_bf16(x):
    u = lax.bitcast_convert_type(x, jnp.uint32)
    r = (u + jnp.uint32(0x7FFF) + ((u >> 16) & jnp.uint32(1))) & jnp.uint32(0xFFFF0000)
    return lax.bitcast_convert_type(r, jnp.float32)


def _sc_top3_body(px_hbm, py_hbm, pz_hbm, s2_hbm, out_hbm,
                  px_v, py_v, pz_v, s2_v, ob_v):
    wid = lax.axis_index("s") * 2 + lax.axis_index("c")
    b = wid // 2
    row0 = jnp.where(wid % 2 == 0, 0, RPW)

    pltpu.sync_copy(px_hbm.at[pl.ds(b * N, N)], px_v)
    pltpu.sync_copy(py_hbm.at[pl.ds(b * N, N)], py_v)
    pltpu.sync_copy(pz_hbm.at[pl.ds(b * N, N)], pz_v)
    pltpu.sync_copy(s2_hbm.at[pl.ds(b * N, N)], s2_v)

    inf16 = jnp.full((L,), _INF, jnp.float32)
    lane = lax.iota(jnp.int32, L)

    def group_body(g, _):
        base = row0 + g * (G * L)
        n2x = [px_v[pl.ds(base + k * L, L)] * -2.0 for k in range(G)]
        n2y = [py_v[pl.ds(base + k * L, L)] * -2.0 for k in range(G)]
        n2z = [pz_v[pl.ds(base + k * L, L)] * -2.0 for k in range(G)]
        riv = [base + k * L + lane for k in range(G)]

        def col_body(j, carry):
            ms, jv = carry
            a = plsc.load_gather(px_v, [jv])
            bb = plsc.load_gather(py_v, [jv])
            c = plsc.load_gather(pz_v, [jv])
            sj = plsc.load_gather(s2_v, [jv])
            out = []
            for k in range(G):
                m1, m2, m3 = ms[k]
                t = sj + a * n2x[k]
                t = t + bb * n2y[k]
                t = t + c * n2z[k]
                t = jnp.where(jv == riv[k], _INF, t)
                h = jnp.maximum(t, m1)
                m1 = jnp.minimum(t, m1)
                h2 = jnp.maximum(h, m2)
                m2 = jnp.minimum(h, m2)
                m3 = jnp.minimum(h2, m3)
                out.append((m1, m2, m3))
            return tuple(out), jv + 1

        zero = jnp.zeros((L,), jnp.int32)
        init = tuple((inf16, inf16, inf16) for _ in range(G))
        ms, _ = lax.fori_loop(0, N, col_body, (init, zero))

        for k in range(G):
            m1, m2, m3 = ms[k]
            sr = s2_v[pl.ds(base + k * L, L)]
            loc = g * (G * L) + k * L
            ob_v[pl.ds(loc, L)] = m1 + sr
            ob_v[pl.ds(RPW + loc, L)] = m2 + sr
            ob_v[pl.ds(2 * RPW + loc, L)] = m3 + sr
        return 0

    lax.fori_loop(0, RPW // (G * L), group_body, 0)

    for kk in range(3):
        pltpu.sync_copy(ob_v.at[pl.ds(kk * RPW, RPW)],
                        out_hbm.at[pl.ds(kk * B * N + b * N + row0, RPW)])


@functools.partial(
    pl.kernel,
    mesh=plsc.VectorSubcoreMesh(core_axis_name="c", subcore_axis_name="s"),
    out_type=jax.ShapeDtypeStruct((3 * B * N,), jnp.float32),
    scratch_types=[
        pltpu.VMEM((N,), jnp.float32),
        pltpu.VMEM((N,), jnp.float32),
        pltpu.VMEM((N,), jnp.float32),
        pltpu.VMEM((N,), jnp.float32),
        pltpu.VMEM((3 * RPW,), jnp.float32),
    ],
    compiler_params=pltpu.CompilerParams(needs_layout_passes=False),
)
def _sc_top3(px_hbm, py_hbm, pz_hbm, s2_hbm, out_hbm,
             px_v, py_v, pz_v, s2_v, ob_v):
    _sc_top3_body(px_hbm, py_hbm, pz_hbm, s2_hbm, out_hbm,
                  px_v, py_v, pz_v, s2_v, ob_v)


def _tc_body(pts_ref, ptsT_ref, top_ref, wrel_ref, brel_ref, wdist_ref,
             bdist_ref, wdens_ref, bdens_ref, wout_ref, bout_ref, out_ref):
    pts_blk = pts_ref[0]
    ptsT = ptsT_ref[0]

    csum = jnp.sum(ptsT, axis=1)
    centroid = (csum / jnp.float32(N))[None, :]
    rel = pts_blk - centroid

    rel_f = lax.dot_general(rel, wrel_ref[...],
                            (((1,), (0,)), ((), ())),
                            preferred_element_type=jnp.float32)
    rel_f = rel_f + brel_ref[...]

    cdist = jnp.sqrt(jnp.sum(rel * rel, axis=1, keepdims=True))
    dist_f = cdist * wdist_ref[...] + bdist_ref[...]

    t = top_ref[0]
    v1 = jnp.maximum(t[:, 0:1], 0.0)
    v2 = jnp.maximum(t[:, 1:2], 0.0)
    v3 = jnp.maximum(t[:, 2:3], 0.0)
    density = (jnp.sqrt(v1) + jnp.sqrt(v2) + jnp.sqrt(v3)) / 3.0

    dens_f = density * wdens_ref[...] + bdens_ref[...]

    feat = jnp.concatenate([rel_f, dist_f, dens_f], axis=1)
    out = lax.dot_general(feat, wout_ref[...],
                          (((1,), (0,)), ((), ())),
                          preferred_element_type=jnp.float32)
    out_ref[0] = out + bout_ref[...]


@jax.jit
def kernel(points, W_rel, b_rel, W_dist, b_dist, W_dens, b_dens, W_out, b_out):
    pointsT = jnp.transpose(points, (0, 2, 1))
    pointsTb = _round_to_bf16(pointsT)
    px = pointsTb[:, 0, :].reshape(B * N)
    py = pointsTb[:, 1, :].reshape(B * N)
    pz = pointsTb[:, 2, :].reshape(B * N)
    s2 = jnp.sum(pointsT * pointsT, axis=1).reshape(B * N)
    top3 = _sc_top3(px, py, pz, s2).reshape(3, B, N)
    top3 = jnp.transpose(top3, (1, 2, 0))
    grid = (B, N // RB)
    out = pl.pallas_call(
        _tc_body,
        grid=grid,
        in_specs=[
            pl.BlockSpec((1, RB, DIM), lambda b, r: (b, r, 0)),
            pl.BlockSpec((1, DIM, N), lambda b, r: (b, 0, 0)),
            pl.BlockSpec((1, RB, 3), lambda b, r: (b, r, 0)),
            pl.BlockSpec((DIM, SUB), lambda b, r: (0, 0)),
            pl.BlockSpec((1, SUB), lambda b, r: (0, 0)),
            pl.BlockSpec((1, SUB), lambda b, r: (0, 0)),
            pl.BlockSpec((1, SUB), lambda b, r: (0, 0)),
            pl.BlockSpec((1, SUB), lambda b, r: (0, 0)),
            pl.BlockSpec((1, SUB), lambda b, r: (0, 0)),
            pl.BlockSpec((EMBED, EMBED), lambda b, r: (0, 0)),
            pl.BlockSpec((1, EMBED), lambda b, r: (0, 0)),
        ],
        out_specs=pl.BlockSpec((1, RB, EMBED), lambda b, r: (b, r, 0)),
        out_shape=jax.ShapeDtypeStruct((B, N, EMBED), jnp.float32),
        compiler_params=pltpu.CompilerParams(
            dimension_semantics=("parallel", "arbitrary"),
        ),
    )(points, pointsT, top3,
      W_rel, b_rel[None, :], W_dist, b_dist[None, :],
      W_dens, b_dens[None, :], W_out, b_out[None, :])
    return out


# speedup vs baseline: 1.7972x; 1.0913x over previous
import functools

import jax
import jax.numpy as jnp
from jax import lax
from jax.experimental import pallas as pl
from jax.experimental.pallas import tpu as pltpu
from jax.experimental.pallas import tpu_sc as plsc

B, N, DIM = 16, 2048, 3
SUB = 128
EMBED = 3 * SUB
RB = 512
NW = 32
RPW = B * N // NW
L = 16
G = 4

_INF = float("inf")


def _round_to_bf16(x):
    u = lax.bitcast_convert_type(x, jnp.uint32)
    r = (u + jnp.uint32(0x7FFF) + ((u >> 16) & jnp.uint32(1))) & jnp.uint32(0xFFFF0000)
    return lax.bitcast_convert_type(r, jnp.float32)


def _sc_top3_body(px_hbm, py_hbm, pz_hbm, s2_hbm, out_hbm,
                  px_v, py_v, pz_v, s2_v, ob_v):
    wid = lax.axis_index("s") * 2 + lax.axis_index("c")
    b = wid // 2
    row0 = jnp.where(wid % 2 == 0, 0, RPW)

    pltpu.sync_copy(px_hbm.at[pl.ds(b * N, N)], px_v)
    pltpu.sync_copy(py_hbm.at[pl.ds(b * N, N)], py_v)
    pltpu.sync_copy(pz_hbm.at[pl.ds(b * N, N)], pz_v)
    pltpu.sync_copy(s2_hbm.at[pl.ds(b * N, N)], s2_v)

    inf16 = jnp.full((L,), _INF, jnp.float32)
    lane = lax.iota(jnp.int32, L)

    def group_body(g, _):
        base = row0 + g * (G * L)
        n2x = [px_v[pl.ds(base + k * L, L)] * -2.0 for k in range(G)]
        n2y = [py_v[pl.ds(base + k * L, L)] * -2.0 for k in range(G)]
        n2z = [pz_v[pl.ds(base + k * L, L)] * -2.0 for k in range(G)]
        riv = [base + k * L + lane for k in range(G)]

        def col_body(j, carry):
            ms, jv = carry
            a = plsc.load_gather(px_v, [jv])
            bb = plsc.load_gather(py_v, [jv])
            c = plsc.load_gather(pz_v, [jv])
            sj = plsc.load_gather(s2_v, [jv])
            out = []
            for k in range(G):
                m1, m2, m3 = ms[k]
                t = sj + a * n2x[k]
                t = t + bb * n2y[k]
                t = t + c * n2z[k]
                t = jnp.where(jv == riv[k], _INF, t)
                h = jnp.maximum(t, m1)
                m1 = jnp.minimum(t, m1)
                h2 = jnp.maximum(h, m2)
                m2 = jnp.minimum(h, m2)
                m3 = jnp.minimum(h2, m3)
                out.append((m1, m2, m3))
            return tuple(out), jv + 1

        zero = jnp.zeros((L,), jnp.int32)
        init = tuple((inf16, inf16, inf16) for _ in range(G))
        ms, _ = lax.fori_loop(0, N, col_body, (init, zero))

        for k in range(G):
            m1, m2, m3 = ms[k]
            sr = s2_v[pl.ds(base + k * L, L)]
            loc = g * (G * L) + k * L
            ob_v[pl.ds(loc, L)] = m1 + sr
            ob_v[pl.ds(RPW + loc, L)] = m2 + sr
            ob_v[pl.ds(2 * RPW + loc, L)] = m3 + sr
        return 0

    lax.fori_loop(0, RPW // (G * L), group_body, 0)

    for kk in range(3):
        pltpu.sync_copy(ob_v.at[pl.ds(kk * RPW, RPW)],
                        out_hbm.at[pl.ds(kk * B * N + b * N + row0, RPW)])


@functools.partial(
    pl.kernel,
    mesh=plsc.VectorSubcoreMesh(core_axis_name="c", subcore_axis_name="s"),
    out_type=jax.ShapeDtypeStruct((3 * B * N,), jnp.float32),
    scratch_types=[
        pltpu.VMEM((N,), jnp.float32),
        pltpu.VMEM((N,), jnp.float32),
        pltpu.VMEM((N,), jnp.float32),
        pltpu.VMEM((N,), jnp.float32),
        pltpu.VMEM((3 * RPW,), jnp.float32),
    ],
    compiler_params=pltpu.CompilerParams(needs_layout_passes=False),
)
def _sc_top3(px_hbm, py_hbm, pz_hbm, s2_hbm, out_hbm,
             px_v, py_v, pz_v, s2_v, ob_v):
    _sc_top3_body(px_hbm, py_hbm, pz_hbm, s2_hbm, out_hbm,
                  px_v, py_v, pz_v, s2_v, ob_v)


def _tc_body(pts_ref, ptsT_ref, top_ref, wrel_ref, brel_ref, wdist_ref,
             bdist_ref, wdens_ref, bdens_ref, wout_ref, bout_ref, out_ref):
    pts_blk = pts_ref[0]
    ptsT = ptsT_ref[0]

    csum = jnp.sum(ptsT, axis=1)
    centroid = (csum / jnp.float32(N))[None, :]
    rel = pts_blk - centroid

    rel_f = lax.dot_general(rel, wrel_ref[...],
                            (((1,), (0,)), ((), ())),
                            preferred_element_type=jnp.float32)
    rel_f = rel_f + brel_ref[...]

    cdist = jnp.sqrt(jnp.sum(rel * rel, axis=1, keepdims=True))
    dist_f = cdist * wdist_ref[...] + bdist_ref[...]

    t = top_ref[0]
    v1 = jnp.maximum(t[:, 0:1], 0.0)
    v2 = jnp.maximum(t[:, 1:2], 0.0)
    v3 = jnp.maximum(t[:, 2:3], 0.0)
    density = (jnp.sqrt(v1) + jnp.sqrt(v2) + jnp.sqrt(v3)) / 3.0

    dens_f = density * wdens_ref[...] + bdens_ref[...]

    feat = jnp.concatenate([rel_f, dist_f, dens_f], axis=1)
    out = lax.dot_general(feat, wout_ref[...],
                          (((1,), (0,)), ((), ())),
                          preferred_element_type=jnp.float32)
    out_ref[0] = out + bout_ref[...]


@jax.jit
def kernel(points, W_rel, b_rel, W_dist, b_dist, W_dens, b_dens, W_out, b_out):
    pointsT = jnp.transpose(points, (0, 2, 1))
    pointsTb = _round_to_bf16(pointsT)
    px = pointsTb[:, 0, :].reshape(B * N)
    py = pointsTb[:, 1, :].reshape(B * N)
    pz = pointsTb[:, 2, :].reshape(B * N)
    s2 = jnp.sum(pointsT * pointsT, axis=1).reshape(B * N)
    top3 = _sc_top3(px, py, pz, s2).reshape(3, B, N)
    top3 = jnp.transpose(top3, (1, 2, 0))
    grid = (B, N // RB)
    out = pl.pallas_call(
        _tc_body,
        grid=grid,
        in_specs=[
            pl.BlockSpec((1, RB, DIM), lambda b, r: (b, r, 0)),
            pl.BlockSpec((1, DIM, N), lambda b, r: (b, 0, 0)),
            pl.BlockSpec((1, RB, 3), lambda b, r: (b, r, 0)),
            pl.BlockSpec((DIM, SUB), lambda b, r: (0, 0)),
            pl.BlockSpec((1, SUB), lambda b, r: (0, 0)),
            pl.BlockSpec((1, SUB), lambda b, r: (0, 0)),
            pl.BlockSpec((1, SUB), lambda b, r: (0, 0)),
            pl.BlockSpec((1, SUB), lambda b, r: (0, 0)),
            pl.BlockSpec((1, SUB), lambda b, r: (0, 0)),
            pl.BlockSpec((EMBED, EMBED), lambda b, r: (0, 0)),
            pl.BlockSpec((1, EMBED), lambda b, r: (0, 0)),
        ],
        out_specs=pl.BlockSpec((1, RB, EMBED), lambda b, r: (b, r, 0)),
        out_shape=jax.ShapeDtypeStruct((B, N, EMBED), jnp.float32),
        compiler_params=pltpu.CompilerParams(
            dimension_semantics=("parallel", "arbitrary"),
        ),
    )(points, pointsT, top3,
      W_rel, b_rel[None, :], W_dist, b_dist[None, :],
      W_dens, b_dens[None, :], W_out, b_out[None, :])
    return out


# speedup vs baseline: 2.1248x; 1.1823x over previous
import functools

import jax
import jax.numpy as jnp
from jax import lax
from jax.experimental import pallas as pl
from jax.experimental.pallas import tpu as pltpu
from jax.experimental.pallas import tpu_sc as plsc

B, N, DIM = 16, 2048, 3
SUB = 128
EMBED = 3 * SUB
RB = 512
NW = 32
RPW = B * N // NW
L = 16
G = 4

_INF = float("inf")


def _round_to_bf16(x):
    u = lax.bitcast_convert_type(x, jnp.uint32)
    r = (u + jnp.uint32(0x7FFF) + ((u >> 16) & jnp.uint32(1))) & jnp.uint32(0xFFFF0000)
    return lax.bitcast_convert_type(r, jnp.float32)


def _sc_top3_body(px_hbm, py_hbm, pz_hbm, s2_hbm, out_hbm,
                  px_v, py_v, pz_v, s2_v, ob_v):
    wid = lax.axis_index("s") * 2 + lax.axis_index("c")
    b = wid // 2
    row0 = jnp.where(wid % 2 == 0, 0, RPW)

    pltpu.sync_copy(px_hbm.at[pl.ds(b * N, N)], px_v)
    pltpu.sync_copy(py_hbm.at[pl.ds(b * N, N)], py_v)
    pltpu.sync_copy(pz_hbm.at[pl.ds(b * N, N)], pz_v)
    pltpu.sync_copy(s2_hbm.at[pl.ds(b * N, N)], s2_v)

    inf16 = jnp.full((L,), _INF, jnp.float32)
    lane = lax.iota(jnp.int32, L)

    def group_body(g, _):
        base = row0 + g * (G * L)
        n2x = [px_v[pl.ds(base + k * L, L)] * -2.0 for k in range(G)]
        n2y = [py_v[pl.ds(base + k * L, L)] * -2.0 for k in range(G)]
        n2z = [pz_v[pl.ds(base + k * L, L)] * -2.0 for k in range(G)]
        riv = [base + k * L + lane for k in range(G)]

        def make_body(with_diag):
            def col_body(j, carry):
                ms, jv = carry
                a = plsc.load_gather(px_v, [jv])
                bb = plsc.load_gather(py_v, [jv])
                c = plsc.load_gather(pz_v, [jv])
                sj = plsc.load_gather(s2_v, [jv])
                out = []
                for k in range(G):
                    m1, m2, m3 = ms[k]
                    t = sj + a * n2x[k]
                    t = t + bb * n2y[k]
                    t = t + c * n2z[k]
                    if with_diag:
                        t = jnp.where(jv == riv[k], _INF, t)
                    h = jnp.maximum(t, m1)
                    m1 = jnp.minimum(t, m1)
                    h2 = jnp.maximum(h, m2)
                    m2 = jnp.minimum(h, m2)
                    m3 = jnp.minimum(h2, m3)
                    out.append((m1, m2, m3))
                return tuple(out), jv + 1
            return col_body

        init = tuple((inf16, inf16, inf16) for _ in range(G))
        jv0 = jnp.zeros((L,), jnp.int32)
        ms, jvb = lax.fori_loop(0, base, make_body(False), (init, jv0))
        ms, jvc = lax.fori_loop(base, base + G * L, make_body(True), (ms, jvb))
        ms, _ = lax.fori_loop(base + G * L, N, make_body(False), (ms, jvc))

        for k in range(G):
            m1, m2, m3 = ms[k]
            sr = s2_v[pl.ds(base + k * L, L)]
            loc = g * (G * L) + k * L
            ob_v[pl.ds(loc, L)] = m1 + sr
            ob_v[pl.ds(RPW + loc, L)] = m2 + sr
            ob_v[pl.ds(2 * RPW + loc, L)] = m3 + sr
        return 0

    lax.fori_loop(0, RPW // (G * L), group_body, 0)

    for kk in range(3):
        pltpu.sync_copy(ob_v.at[pl.ds(kk * RPW, RPW)],
                        out_hbm.at[pl.ds(kk * B * N + b * N + row0, RPW)])


@functools.partial(
    pl.kernel,
    mesh=plsc.VectorSubcoreMesh(core_axis_name="c", subcore_axis_name="s"),
    out_type=jax.ShapeDtypeStruct((3 * B * N,), jnp.float32),
    scratch_types=[
        pltpu.VMEM((N,), jnp.float32),
        pltpu.VMEM((N,), jnp.float32),
        pltpu.VMEM((N,), jnp.float32),
        pltpu.VMEM((N,), jnp.float32),
        pltpu.VMEM((3 * RPW,), jnp.float32),
    ],
    compiler_params=pltpu.CompilerParams(needs_layout_passes=False),
)
def _sc_top3(px_hbm, py_hbm, pz_hbm, s2_hbm, out_hbm,
             px_v, py_v, pz_v, s2_v, ob_v):
    _sc_top3_body(px_hbm, py_hbm, pz_hbm, s2_hbm, out_hbm,
                  px_v, py_v, pz_v, s2_v, ob_v)


def _tc_body(pts_ref, ptsT_ref, top_ref, wrel_ref, brel_ref, wdist_ref,
             bdist_ref, wdens_ref, bdens_ref, wout_ref, bout_ref, out_ref):
    pts_blk = pts_ref[0]
    ptsT = ptsT_ref[0]

    csum = jnp.sum(ptsT, axis=1)
    centroid = (csum / jnp.float32(N))[None, :]
    rel = pts_blk - centroid

    rel_f = lax.dot_general(rel, wrel_ref[...],
                            (((1,), (0,)), ((), ())),
                            preferred_element_type=jnp.float32)
    rel_f = rel_f + brel_ref[...]

    cdist = jnp.sqrt(jnp.sum(rel * rel, axis=1, keepdims=True))
    dist_f = cdist * wdist_ref[...] + bdist_ref[...]

    t = top_ref[0]
    v1 = jnp.maximum(t[:, 0:1], 0.0)
    v2 = jnp.maximum(t[:, 1:2], 0.0)
    v3 = jnp.maximum(t[:, 2:3], 0.0)
    density = (jnp.sqrt(v1) + jnp.sqrt(v2) + jnp.sqrt(v3)) / 3.0

    dens_f = density * wdens_ref[...] + bdens_ref[...]

    feat = jnp.concatenate([rel_f, dist_f, dens_f], axis=1)
    out = lax.dot_general(feat, wout_ref[...],
                          (((1,), (0,)), ((), ())),
                          preferred_element_type=jnp.float32)
    out_ref[0] = out + bout_ref[...]


@jax.jit
def kernel(points, W_rel, b_rel, W_dist, b_dist, W_dens, b_dens, W_out, b_out):
    pointsT = jnp.transpose(points, (0, 2, 1))
    pointsTb = _round_to_bf16(pointsT)
    px = pointsTb[:, 0, :].reshape(B * N)
    py = pointsTb[:, 1, :].reshape(B * N)
    pz = pointsTb[:, 2, :].reshape(B * N)
    s2 = jnp.sum(pointsT * pointsT, axis=1).reshape(B * N)
    top3 = _sc_top3(px, py, pz, s2).reshape(3, B, N)
    top3 = jnp.transpose(top3, (1, 2, 0))
    grid = (B, N // RB)
    out = pl.pallas_call(
        _tc_body,
        grid=grid,
        in_specs=[
            pl.BlockSpec((1, RB, DIM), lambda b, r: (b, r, 0)),
            pl.BlockSpec((1, DIM, N), lambda b, r: (b, 0, 0)),
            pl.BlockSpec((1, RB, 3), lambda b, r: (b, r, 0)),
            pl.BlockSpec((DIM, SUB), lambda b, r: (0, 0)),
            pl.BlockSpec((1, SUB), lambda b, r: (0, 0)),
            pl.BlockSpec((1, SUB), lambda b, r: (0, 0)),
            pl.BlockSpec((1, SUB), lambda b, r: (0, 0)),
            pl.BlockSpec((1, SUB), lambda b, r: (0, 0)),
            pl.BlockSpec((1, SUB), lambda b, r: (0, 0)),
            pl.BlockSpec((EMBED, EMBED), lambda b, r: (0, 0)),
            pl.BlockSpec((1, EMBED), lambda b, r: (0, 0)),
        ],
        out_specs=pl.BlockSpec((1, RB, EMBED), lambda b, r: (b, r, 0)),
        out_shape=jax.ShapeDtypeStruct((B, N, EMBED), jnp.float32),
        compiler_params=pltpu.CompilerParams(
            dimension_semantics=("parallel", "arbitrary"),
        ),
    )(points, pointsT, top3,
      W_rel, b_rel[None, :], W_dist, b_dist[None, :],
      W_dens, b_dens[None, :], W_out, b_out[None, :])
    return out


# speedup vs baseline: 2.2741x; 1.0702x over previous
import functools

import jax
import jax.numpy as jnp
from jax import lax
from jax.experimental import pallas as pl
from jax.experimental.pallas import tpu as pltpu
from jax.experimental.pallas import tpu_sc as plsc

B, N, DIM = 16, 2048, 3
SUB = 128
EMBED = 3 * SUB
RB = 1024
NW = 32
RPW = B * N // NW
L = 16
G = 4

_INF = float("inf")


def _round_to_bf16(x):
    u = lax.bitcast_convert_type(x, jnp.uint32)
    r = (u + jnp.uint32(0x7FFF) + ((u >> 16) & jnp.uint32(1))) & jnp.uint32(0xFFFF0000)
    return lax.bitcast_convert_type(r, jnp.float32)


def _sc_top3_body(px_hbm, py_hbm, pz_hbm, s2_hbm, out_hbm,
                  px_v, py_v, pz_v, s2_v, ob_v):
    wid = lax.axis_index("s") * 2 + lax.axis_index("c")
    b = wid // 2
    row0 = jnp.where(wid % 2 == 0, 0, RPW)

    pltpu.sync_copy(px_hbm.at[pl.ds(b * N, N)], px_v)
    pltpu.sync_copy(py_hbm.at[pl.ds(b * N, N)], py_v)
    pltpu.sync_copy(pz_hbm.at[pl.ds(b * N, N)], pz_v)
    pltpu.sync_copy(s2_hbm.at[pl.ds(b * N, N)], s2_v)

    inf16 = jnp.full((L,), _INF, jnp.float32)
    lane = lax.iota(jnp.int32, L)

    def group_body(g, _):
        base = row0 + g * (G * L)
        n2x = [px_v[pl.ds(base + k * L, L)] * -2.0 for k in range(G)]
        n2y = [py_v[pl.ds(base + k * L, L)] * -2.0 for k in range(G)]
        n2z = [pz_v[pl.ds(base + k * L, L)] * -2.0 for k in range(G)]
        riv = [base + k * L + lane for k in range(G)]

        def make_body(with_diag):
            def col_body(j, carry):
                ms, jv = carry
                a = plsc.load_gather(px_v, [jv])
                bb = plsc.load_gather(py_v, [jv])
                c = plsc.load_gather(pz_v, [jv])
                sj = plsc.load_gather(s2_v, [jv])
                out = []
                for k in range(G):
                    m1, m2, m3 = ms[k]
                    t = sj + a * n2x[k]
                    t = t + bb * n2y[k]
                    t = t + c * n2z[k]
                    if with_diag:
                        t = jnp.where(jv == riv[k], _INF, t)
                    h = jnp.maximum(t, m1)
                    m1 = jnp.minimum(t, m1)
                    h2 = jnp.maximum(h, m2)
                    m2 = jnp.minimum(h, m2)
                    m3 = jnp.minimum(h2, m3)
                    out.append((m1, m2, m3))
                return tuple(out), jv + 1
            return col_body

        init = tuple((inf16, inf16, inf16) for _ in range(G))
        jv0 = jnp.zeros((L,), jnp.int32)
        ms, jvb = lax.fori_loop(0, base, make_body(False), (init, jv0))
        ms, jvc = lax.fori_loop(base, base + G * L, make_body(True), (ms, jvb))
        ms, _ = lax.fori_loop(base + G * L, N, make_body(False), (ms, jvc))

        for k in range(G):
            m1, m2, m3 = ms[k]
            sr = s2_v[pl.ds(base + k * L, L)]
            loc = g * (G * L) + k * L
            ob_v[pl.ds(loc, L)] = m1 + sr
            ob_v[pl.ds(RPW + loc, L)] = m2 + sr
            ob_v[pl.ds(2 * RPW + loc, L)] = m3 + sr
        return 0

    lax.fori_loop(0, RPW // (G * L), group_body, 0)

    for kk in range(3):
        pltpu.sync_copy(ob_v.at[pl.ds(kk * RPW, RPW)],
                        out_hbm.at[pl.ds(kk * B * N + b * N + row0, RPW)])


@functools.partial(
    pl.kernel,
    mesh=plsc.VectorSubcoreMesh(core_axis_name="c", subcore_axis_name="s"),
    out_type=jax.ShapeDtypeStruct((3 * B * N,), jnp.float32),
    scratch_types=[
        pltpu.VMEM((N,), jnp.float32),
        pltpu.VMEM((N,), jnp.float32),
        pltpu.VMEM((N,), jnp.float32),
        pltpu.VMEM((N,), jnp.float32),
        pltpu.VMEM((3 * RPW,), jnp.float32),
    ],
    compiler_params=pltpu.CompilerParams(needs_layout_passes=False),
)
def _sc_top3(px_hbm, py_hbm, pz_hbm, s2_hbm, out_hbm,
             px_v, py_v, pz_v, s2_v, ob_v):
    _sc_top3_body(px_hbm, py_hbm, pz_hbm, s2_hbm, out_hbm,
                  px_v, py_v, pz_v, s2_v, ob_v)


def _tc_body(pts_ref, ptsT_ref, top_ref, wrel_ref, brel_ref, wdist_ref,
             bdist_ref, wdens_ref, bdens_ref, wout_ref, bout_ref, out_ref):
    pts_blk = pts_ref[0]
    ptsT = ptsT_ref[0]

    csum = jnp.sum(ptsT, axis=1)
    centroid = (csum / jnp.float32(N))[None, :]
    rel = pts_blk - centroid

    rel_f = lax.dot_general(rel, wrel_ref[...],
                            (((1,), (0,)), ((), ())),
                            preferred_element_type=jnp.float32)
    rel_f = rel_f + brel_ref[...]

    cdist = jnp.sqrt(jnp.sum(rel * rel, axis=1, keepdims=True))
    dist_f = cdist * wdist_ref[...] + bdist_ref[...]

    t = top_ref[0]
    v1 = jnp.maximum(t[:, 0:1], 0.0)
    v2 = jnp.maximum(t[:, 1:2], 0.0)
    v3 = jnp.maximum(t[:, 2:3], 0.0)
    density = (jnp.sqrt(v1) + jnp.sqrt(v2) + jnp.sqrt(v3)) / 3.0

    dens_f = density * wdens_ref[...] + bdens_ref[...]

    feat = jnp.concatenate([rel_f, dist_f, dens_f], axis=1)
    out = lax.dot_general(feat, wout_ref[...],
                          (((1,), (0,)), ((), ())),
                          preferred_element_type=jnp.float32)
    out_ref[0] = out + bout_ref[...]


@jax.jit
def kernel(points, W_rel, b_rel, W_dist, b_dist, W_dens, b_dens, W_out, b_out):
    pointsT = jnp.transpose(points, (0, 2, 1))
    pointsTb = _round_to_bf16(pointsT)
    px = pointsTb[:, 0, :].reshape(B * N)
    py = pointsTb[:, 1, :].reshape(B * N)
    pz = pointsTb[:, 2, :].reshape(B * N)
    s2 = jnp.sum(pointsT * pointsT, axis=1).reshape(B * N)
    top3 = _sc_top3(px, py, pz, s2).reshape(3, B, N)
    top3 = jnp.transpose(top3, (1, 2, 0))
    grid = (B, N // RB)
    out = pl.pallas_call(
        _tc_body,
        grid=grid,
        in_specs=[
            pl.BlockSpec((1, RB, DIM), lambda b, r: (b, r, 0)),
            pl.BlockSpec((1, DIM, N), lambda b, r: (b, 0, 0)),
            pl.BlockSpec((1, RB, 3), lambda b, r: (b, r, 0)),
            pl.BlockSpec((DIM, SUB), lambda b, r: (0, 0)),
            pl.BlockSpec((1, SUB), lambda b, r: (0, 0)),
            pl.BlockSpec((1, SUB), lambda b, r: (0, 0)),
            pl.BlockSpec((1, SUB), lambda b, r: (0, 0)),
            pl.BlockSpec((1, SUB), lambda b, r: (0, 0)),
            pl.BlockSpec((1, SUB), lambda b, r: (0, 0)),
            pl.BlockSpec((EMBED, EMBED), lambda b, r: (0, 0)),
            pl.BlockSpec((1, EMBED), lambda b, r: (0, 0)),
        ],
        out_specs=pl.BlockSpec((1, RB, EMBED), lambda b, r: (b, r, 0)),
        out_shape=jax.ShapeDtypeStruct((B, N, EMBED), jnp.float32),
        compiler_params=pltpu.CompilerParams(
            dimension_semantics=("parallel", "arbitrary"),
        ),
    )(points, pointsT, top3,
      W_rel, b_rel[None, :], W_dist, b_dist[None, :],
      W_dens, b_dens[None, :], W_out, b_out[None, :])
    return out


# speedup vs baseline: 3.4136x; 1.5011x over previous
import functools

import jax
import jax.numpy as jnp
from jax import lax
from jax.experimental import pallas as pl
from jax.experimental.pallas import tpu as pltpu
from jax.experimental.pallas import tpu_sc as plsc

B, N, DIM = 16, 2048, 3
SUB = 128
EMBED = 3 * SUB
RB = 1024
NW = 32
BS = 4
B2 = B - BS
WPB = NW // BS
RPW = N // WPB
L = 16
G = 4
RB2 = 256

_INF = float("inf")


def _round_to_bf16(x):
    u = lax.bitcast_convert_type(x, jnp.uint32)
    r = (u + jnp.uint32(0x7FFF) + ((u >> 16) & jnp.uint32(1))) & jnp.uint32(0xFFFF0000)
    return lax.bitcast_convert_type(r, jnp.float32)


def _sc_top3_body(px_hbm, py_hbm, pz_hbm, s2_hbm, out_hbm,
                  px_v, py_v, pz_v, s2_v, ob_v):
    wid = lax.axis_index("s") * 2 + lax.axis_index("c")
    b = wid // WPB
    row0 = (wid % WPB) * RPW

    pltpu.sync_copy(px_hbm.at[pl.ds(b * N, N)], px_v)
    pltpu.sync_copy(py_hbm.at[pl.ds(b * N, N)], py_v)
    pltpu.sync_copy(pz_hbm.at[pl.ds(b * N, N)], pz_v)
    pltpu.sync_copy(s2_hbm.at[pl.ds(b * N, N)], s2_v)

    inf16 = jnp.full((L,), _INF, jnp.float32)
    lane = lax.iota(jnp.int32, L)

    def group_body(g, _):
        base = row0 + g * (G * L)
        n2x = [px_v[pl.ds(base + k * L, L)] * -2.0 for k in range(G)]
        n2y = [py_v[pl.ds(base + k * L, L)] * -2.0 for k in range(G)]
        n2z = [pz_v[pl.ds(base + k * L, L)] * -2.0 for k in range(G)]
        riv = [base + k * L + lane for k in range(G)]

        def make_body(with_diag):
            def col_body(j, carry):
                ms, jv = carry
                a = plsc.load_gather(px_v, [jv])
                bb = plsc.load_gather(py_v, [jv])
                c = plsc.load_gather(pz_v, [jv])
                sj = plsc.load_gather(s2_v, [jv])
                out = []
                for k in range(G):
                    m1, m2, m3 = ms[k]
                    t = sj + a * n2x[k]
                    t = t + bb * n2y[k]
                    t = t + c * n2z[k]
                    if with_diag:
                        t = jnp.where(jv == riv[k], _INF, t)
                    h = jnp.maximum(t, m1)
                    m1 = jnp.minimum(t, m1)
                    h2 = jnp.maximum(h, m2)
                    m2 = jnp.minimum(h, m2)
                    m3 = jnp.minimum(h2, m3)
                    out.append((m1, m2, m3))
                return tuple(out), jv + 1
            return col_body

        init = tuple((inf16, inf16, inf16) for _ in range(G))
        jv0 = jnp.zeros((L,), jnp.int32)
        ms, jvb = lax.fori_loop(0, base, make_body(False), (init, jv0))
        ms, jvc = lax.fori_loop(base, base + G * L, make_body(True), (ms, jvb))
        ms, _ = lax.fori_loop(base + G * L, N, make_body(False), (ms, jvc))

        for k in range(G):
            m1, m2, m3 = ms[k]
            sr = s2_v[pl.ds(base + k * L, L)]
            loc = g * (G * L) + k * L
            ob_v[pl.ds(loc, L)] = m1 + sr
            ob_v[pl.ds(RPW + loc, L)] = m2 + sr
            ob_v[pl.ds(2 * RPW + loc, L)] = m3 + sr
        return 0

    lax.fori_loop(0, RPW // (G * L), group_body, 0)

    for kk in range(3):
        pltpu.sync_copy(ob_v.at[pl.ds(kk * RPW, RPW)],
                        out_hbm.at[pl.ds(kk * BS * N + b * N + row0, RPW)])


@functools.partial(
    pl.kernel,
    mesh=plsc.VectorSubcoreMesh(core_axis_name="c", subcore_axis_name="s"),
    out_type=jax.ShapeDtypeStruct((3 * BS * N,), jnp.float32),
    scratch_types=[
        pltpu.VMEM((N,), jnp.float32),
        pltpu.VMEM((N,), jnp.float32),
        pltpu.VMEM((N,), jnp.float32),
        pltpu.VMEM((N,), jnp.float32),
        pltpu.VMEM((3 * RPW,), jnp.float32),
    ],
    compiler_params=pltpu.CompilerParams(needs_layout_passes=False),
)
def _sc_top3(px_hbm, py_hbm, pz_hbm, s2_hbm, out_hbm,
             px_v, py_v, pz_v, s2_v, ob_v):
    _sc_top3_body(px_hbm, py_hbm, pz_hbm, s2_hbm, out_hbm,
                  px_v, py_v, pz_v, s2_v, ob_v)


def _tc_top3_body(pts_ref, ptsT_ref, out_ref):
    rb = pl.program_id(1)
    pts_blk = pts_ref[0]
    ptsT = ptsT_ref[0]

    x2r = jnp.sum(pts_blk * pts_blk, axis=1, keepdims=True)
    x2c = jnp.sum(ptsT * ptsT, axis=0, keepdims=True)
    g = lax.dot_general(pts_blk, ptsT,
                        (((1,), (0,)), ((), ())),
                        preferred_element_type=jnp.float32)
    d2 = jnp.maximum(x2r + x2c - 2.0 * g, 0.0)

    row_ids = rb * RB2 + lax.broadcasted_iota(jnp.int32, (RB2, 1), 0)
    col_ids = lax.broadcasted_iota(jnp.int32, (1, N), 1)
    d2 = jnp.where(row_ids == col_ids, _INF, d2)

    m1 = jnp.min(d2, axis=1, keepdims=True)
    le1 = d2 <= m1
    c1 = jnp.sum(le1.astype(jnp.float32), axis=1, keepdims=True)
    d2b = jnp.where(le1, _INF, d2)
    m2 = jnp.min(d2b, axis=1, keepdims=True)
    le2 = d2b <= m2
    c2 = jnp.sum(le2.astype(jnp.float32), axis=1, keepdims=True)
    d2c = jnp.where(le2, _INF, d2b)
    m3 = jnp.min(d2c, axis=1, keepdims=True)

    out1 = m1
    out2 = jnp.where(c1 >= 2.0, m1, m2)
    out3 = jnp.where(c1 >= 3.0, m1, jnp.where(c1 + c2 >= 3.0, m2, m3))
    out_ref[0] = jnp.concatenate([out1, out2, out3], axis=1)


def _tc_top3(pts2, pts2T):
    grid = (B2, N // RB2)
    return pl.pallas_call(
        _tc_top3_body,
        grid=grid,
        in_specs=[
            pl.BlockSpec((1, RB2, DIM), lambda b, r: (b, r, 0)),
            pl.BlockSpec((1, DIM, N), lambda b, r: (b, 0, 0)),
        ],
        out_specs=pl.BlockSpec((1, RB2, 3), lambda b, r: (b, r, 0)),
        out_shape=jax.ShapeDtypeStruct((B2, N, 3), jnp.float32),
        compiler_params=pltpu.CompilerParams(
            dimension_semantics=("parallel", "arbitrary"),
        ),
    )(pts2, pts2T)


def _tc_body(pts_ref, ptsT_ref, top_ref, wrel_ref, brel_ref, wdist_ref,
             bdist_ref, wdens_ref, bdens_ref, wout_ref, bout_ref, out_ref):
    pts_blk = pts_ref[0]
    ptsT = ptsT_ref[0]

    csum = jnp.sum(ptsT, axis=1)
    centroid = (csum / jnp.float32(N))[None, :]
    rel = pts_blk - centroid

    rel_f = lax.dot_general(rel, wrel_ref[...],
                            (((1,), (0,)), ((), ())),
                            preferred_element_type=jnp.float32)
    rel_f = rel_f + brel_ref[...]

    cdist = jnp.sqrt(jnp.sum(rel * rel, axis=1, keepdims=True))
    dist_f = cdist * wdist_ref[...] + bdist_ref[...]

    t = top_ref[0]
    v1 = jnp.maximum(t[:, 0:1], 0.0)
    v2 = jnp.maximum(t[:, 1:2], 0.0)
    v3 = jnp.maximum(t[:, 2:3], 0.0)
    density = (jnp.sqrt(v1) + jnp.sqrt(v2) + jnp.sqrt(v3)) / 3.0

    dens_f = density * wdens_ref[...] + bdens_ref[...]

    feat = jnp.concatenate([rel_f, dist_f, dens_f], axis=1)
    out = lax.dot_general(feat, wout_ref[...],
                          (((1,), (0,)), ((), ())),
                          preferred_element_type=jnp.float32)
    out_ref[0] = out + bout_ref[...]


@jax.jit
def kernel(points, W_rel, b_rel, W_dist, b_dist, W_dens, b_dens, W_out, b_out):
    pointsT = jnp.transpose(points, (0, 2, 1))
    pointsTb = _round_to_bf16(pointsT[:BS])
    px = pointsTb[:, 0, :].reshape(BS * N)
    py = pointsTb[:, 1, :].reshape(BS * N)
    pz = pointsTb[:, 2, :].reshape(BS * N)
    s2 = jnp.sum(pointsT[:BS] * pointsT[:BS], axis=1).reshape(BS * N)
    top3_sc = _sc_top3(px, py, pz, s2).reshape(3, BS, N)
    top3_sc = jnp.transpose(top3_sc, (1, 2, 0))
    top3_tc = _tc_top3(points[BS:], pointsT[BS:])
    top3 = jnp.concatenate([top3_sc, top3_tc], axis=0)
    grid = (B, N // RB)
    out = pl.pallas_call(
        _tc_body,
        grid=grid,
        in_specs=[
            pl.BlockSpec((1, RB, DIM), lambda b, r: (b, r, 0)),
            pl.BlockSpec((1, DIM, N), lambda b, r: (b, 0, 0)),
            pl.BlockSpec((1, RB, 3), lambda b, r: (b, r, 0)),
            pl.BlockSpec((DIM, SUB), lambda b, r: (0, 0)),
            pl.BlockSpec((1, SUB), lambda b, r: (0, 0)),
            pl.BlockSpec((1, SUB), lambda b, r: (0, 0)),
            pl.BlockSpec((1, SUB), lambda b, r: (0, 0)),
            pl.BlockSpec((1, SUB), lambda b, r: (0, 0)),
            pl.BlockSpec((1, SUB), lambda b, r: (0, 0)),
            pl.BlockSpec((EMBED, EMBED), lambda b, r: (0, 0)),
            pl.BlockSpec((1, EMBED), lambda b, r: (0, 0)),
        ],
        out_specs=pl.BlockSpec((1, RB, EMBED), lambda b, r: (b, r, 0)),
        out_shape=jax.ShapeDtypeStruct((B, N, EMBED), jnp.float32),
        compiler_params=pltpu.CompilerParams(
            dimension_semantics=("parallel", "arbitrary"),
        ),
    )(points, pointsT, top3,
      W_rel, b_rel[None, :], W_dist, b_dist[None, :],
      W_dens, b_dens[None, :], W_out, b_out[None, :])
    return out


# speedup vs baseline: 3.5540x; 1.0411x over previous
import functools

import jax
import jax.numpy as jnp
from jax import lax
from jax.experimental import pallas as pl
from jax.experimental.pallas import tpu as pltpu
from jax.experimental.pallas import tpu_sc as plsc

B, N, DIM = 16, 2048, 3
SUB = 128
EMBED = 3 * SUB
RB = 2048
NW = 32
BS = 4
B2 = B - BS
WPB = NW // BS
RPW = N // WPB
L = 16
G = 4
RB2 = 256

_INF = float("inf")


def _round_to_bf16(x):
    u = lax.bitcast_convert_type(x, jnp.uint32)
    r = (u + jnp.uint32(0x7FFF) + ((u >> 16) & jnp.uint32(1))) & jnp.uint32(0xFFFF0000)
    return lax.bitcast_convert_type(r, jnp.float32)


def _sc_top3_body(px_hbm, py_hbm, pz_hbm, s2_hbm, out_hbm,
                  px_v, py_v, pz_v, s2_v, ob_v):
    wid = lax.axis_index("s") * 2 + lax.axis_index("c")
    b = wid // WPB
    row0 = (wid % WPB) * RPW

    pltpu.sync_copy(px_hbm.at[pl.ds(b * N, N)], px_v)
    pltpu.sync_copy(py_hbm.at[pl.ds(b * N, N)], py_v)
    pltpu.sync_copy(pz_hbm.at[pl.ds(b * N, N)], pz_v)
    pltpu.sync_copy(s2_hbm.at[pl.ds(b * N, N)], s2_v)

    inf16 = jnp.full((L,), _INF, jnp.float32)
    lane = lax.iota(jnp.int32, L)

    def group_body(g, _):
        base = row0 + g * (G * L)
        n2x = [px_v[pl.ds(base + k * L, L)] * -2.0 for k in range(G)]
        n2y = [py_v[pl.ds(base + k * L, L)] * -2.0 for k in range(G)]
        n2z = [pz_v[pl.ds(base + k * L, L)] * -2.0 for k in range(G)]
        riv = [base + k * L + lane for k in range(G)]

        def make_body(with_diag):
            def col_body(j, carry):
                ms, jv = carry
                a = plsc.load_gather(px_v, [jv])
                bb = plsc.load_gather(py_v, [jv])
                c = plsc.load_gather(pz_v, [jv])
                sj = plsc.load_gather(s2_v, [jv])
                out = []
                for k in range(G):
                    m1, m2, m3 = ms[k]
                    t = sj + a * n2x[k]
                    t = t + bb * n2y[k]
                    t = t + c * n2z[k]
                    if with_diag:
                        t = jnp.where(jv == riv[k], _INF, t)
                    h = jnp.maximum(t, m1)
                    m1 = jnp.minimum(t, m1)
                    h2 = jnp.maximum(h, m2)
                    m2 = jnp.minimum(h, m2)
                    m3 = jnp.minimum(h2, m3)
                    out.append((m1, m2, m3))
                return tuple(out), jv + 1
            return col_body

        init = tuple((inf16, inf16, inf16) for _ in range(G))
        jv0 = jnp.zeros((L,), jnp.int32)
        ms, jvb = lax.fori_loop(0, base, make_body(False), (init, jv0))
        ms, jvc = lax.fori_loop(base, base + G * L, make_body(True), (ms, jvb))
        ms, _ = lax.fori_loop(base + G * L, N, make_body(False), (ms, jvc))

        for k in range(G):
            m1, m2, m3 = ms[k]
            sr = s2_v[pl.ds(base + k * L, L)]
            loc = g * (G * L) + k * L
            ob_v[pl.ds(loc, L)] = m1 + sr
            ob_v[pl.ds(RPW + loc, L)] = m2 + sr
            ob_v[pl.ds(2 * RPW + loc, L)] = m3 + sr
        return 0

    lax.fori_loop(0, RPW // (G * L), group_body, 0)

    for kk in range(3):
        pltpu.sync_copy(ob_v.at[pl.ds(kk * RPW, RPW)],
                        out_hbm.at[pl.ds(kk * BS * N + b * N + row0, RPW)])


@functools.partial(
    pl.kernel,
    mesh=plsc.VectorSubcoreMesh(core_axis_name="c", subcore_axis_name="s"),
    out_type=jax.ShapeDtypeStruct((3 * BS * N,), jnp.float32),
    scratch_types=[
        pltpu.VMEM((N,), jnp.float32),
        pltpu.VMEM((N,), jnp.float32),
        pltpu.VMEM((N,), jnp.float32),
        pltpu.VMEM((N,), jnp.float32),
        pltpu.VMEM((3 * RPW,), jnp.float32),
    ],
    compiler_params=pltpu.CompilerParams(needs_layout_passes=False),
)
def _sc_top3(px_hbm, py_hbm, pz_hbm, s2_hbm, out_hbm,
             px_v, py_v, pz_v, s2_v, ob_v):
    _sc_top3_body(px_hbm, py_hbm, pz_hbm, s2_hbm, out_hbm,
                  px_v, py_v, pz_v, s2_v, ob_v)


def _tc_top3_body(pts_ref, ptsT_ref, out_ref):
    rb = pl.program_id(1)
    pts_blk = pts_ref[0]
    ptsT = ptsT_ref[0]

    x2r = jnp.sum(pts_blk * pts_blk, axis=1, keepdims=True)
    x2c = jnp.sum(ptsT * ptsT, axis=0, keepdims=True)
    g = lax.dot_general(pts_blk, ptsT,
                        (((1,), (0,)), ((), ())),
                        preferred_element_type=jnp.float32)
    d2 = jnp.maximum(x2r + x2c - 2.0 * g, 0.0)

    row_ids = rb * RB2 + lax.broadcasted_iota(jnp.int32, (RB2, 1), 0)
    col_ids = lax.broadcasted_iota(jnp.int32, (1, N), 1)
    d2 = jnp.where(row_ids == col_ids, _INF, d2)

    m1 = jnp.min(d2, axis=1, keepdims=True)
    le1 = d2 <= m1
    c1 = jnp.sum(le1.astype(jnp.float32), axis=1, keepdims=True)
    d2b = jnp.where(le1, _INF, d2)
    m2 = jnp.min(d2b, axis=1, keepdims=True)
    le2 = d2b <= m2
    c2 = jnp.sum(le2.astype(jnp.float32), axis=1, keepdims=True)
    d2c = jnp.where(le2, _INF, d2b)
    m3 = jnp.min(d2c, axis=1, keepdims=True)

    out1 = m1
    out2 = jnp.where(c1 >= 2.0, m1, m2)
    out3 = jnp.where(c1 >= 3.0, m1, jnp.where(c1 + c2 >= 3.0, m2, m3))
    out_ref[0] = jnp.concatenate([out1, out2, out3], axis=1)


def _tc_top3(pts2, pts2T):
    grid = (B2, N // RB2)
    return pl.pallas_call(
        _tc_top3_body,
        grid=grid,
        in_specs=[
            pl.BlockSpec((1, RB2, DIM), lambda b, r: (b, r, 0)),
            pl.BlockSpec((1, DIM, N), lambda b, r: (b, 0, 0)),
        ],
        out_specs=pl.BlockSpec((1, RB2, 3), lambda b, r: (b, r, 0)),
        out_shape=jax.ShapeDtypeStruct((B2, N, 3), jnp.float32),
        compiler_params=pltpu.CompilerParams(
            dimension_semantics=("parallel", "arbitrary"),
        ),
    )(pts2, pts2T)


def _tc_body(pts_ref, ptsT_ref, top_ref, wrel_ref, brel_ref, wdist_ref,
             bdist_ref, wdens_ref, bdens_ref, wout_ref, bout_ref, out_ref):
    pts_blk = pts_ref[0]
    ptsT = ptsT_ref[0]

    csum = jnp.sum(ptsT, axis=1)
    centroid = (csum / jnp.float32(N))[None, :]
    rel = pts_blk - centroid

    rel_f = lax.dot_general(rel, wrel_ref[...],
                            (((1,), (0,)), ((), ())),
                            preferred_element_type=jnp.float32)
    rel_f = rel_f + brel_ref[...]

    cdist = jnp.sqrt(jnp.sum(rel * rel, axis=1, keepdims=True))
    dist_f = cdist * wdist_ref[...] + bdist_ref[...]

    t = top_ref[0]
    v1 = jnp.maximum(t[:, 0:1], 0.0)
    v2 = jnp.maximum(t[:, 1:2], 0.0)
    v3 = jnp.maximum(t[:, 2:3], 0.0)
    density = (jnp.sqrt(v1) + jnp.sqrt(v2) + jnp.sqrt(v3)) / 3.0

    dens_f = density * wdens_ref[...] + bdens_ref[...]

    feat = jnp.concatenate([rel_f, dist_f, dens_f], axis=1)
    out = lax.dot_general(feat, wout_ref[...],
                          (((1,), (0,)), ((), ())),
                          preferred_element_type=jnp.float32)
    out_ref[0] = out + bout_ref[...]


@jax.jit
def kernel(points, W_rel, b_rel, W_dist, b_dist, W_dens, b_dens, W_out, b_out):
    pointsT = jnp.transpose(points, (0, 2, 1))
    pointsTb = _round_to_bf16(pointsT[:BS])
    px = pointsTb[:, 0, :].reshape(BS * N)
    py = pointsTb[:, 1, :].reshape(BS * N)
    pz = pointsTb[:, 2, :].reshape(BS * N)
    s2 = jnp.sum(pointsT[:BS] * pointsT[:BS], axis=1).reshape(BS * N)
    top3_sc = _sc_top3(px, py, pz, s2).reshape(3, BS, N)
    top3_sc = jnp.transpose(top3_sc, (1, 2, 0))
    top3_tc = _tc_top3(points[BS:], pointsT[BS:])
    top3 = jnp.concatenate([top3_sc, top3_tc], axis=0)
    grid = (B, N // RB)
    out = pl.pallas_call(
        _tc_body,
        grid=grid,
        in_specs=[
            pl.BlockSpec((1, RB, DIM), lambda b, r: (b, r, 0)),
            pl.BlockSpec((1, DIM, N), lambda b, r: (b, 0, 0)),
            pl.BlockSpec((1, RB, 3), lambda b, r: (b, r, 0)),
            pl.BlockSpec((DIM, SUB), lambda b, r: (0, 0)),
            pl.BlockSpec((1, SUB), lambda b, r: (0, 0)),
            pl.BlockSpec((1, SUB), lambda b, r: (0, 0)),
            pl.BlockSpec((1, SUB), lambda b, r: (0, 0)),
            pl.BlockSpec((1, SUB), lambda b, r: (0, 0)),
            pl.BlockSpec((1, SUB), lambda b, r: (0, 0)),
            pl.BlockSpec((EMBED, EMBED), lambda b, r: (0, 0)),
            pl.BlockSpec((1, EMBED), lambda b, r: (0, 0)),
        ],
        out_specs=pl.BlockSpec((1, RB, EMBED), lambda b, r: (b, r, 0)),
        out_shape=jax.ShapeDtypeStruct((B, N, EMBED), jnp.float32),
        compiler_params=pltpu.CompilerParams(
            dimension_semantics=("parallel", "arbitrary"),
        ),
    )(points, pointsT, top3,
      W_rel, b_rel[None, :], W_dist, b_dist[None, :],
      W_dens, b_dens[None, :], W_out, b_out[None, :])
    return out
